# Initial kernel scaffold; baseline (speedup 1.0000x reference)
import jax
import jax.numpy as jnp
from jax.experimental import pallas as pl


def kernel(input_ids, text_mask, emb_text_table, emb_code_tables):
    raise NotImplementedError("write your pallas kernel here")


# speedup vs baseline: 1.9704x; 1.9704x over previous
import functools

import jax
import jax.numpy as jnp
from jax import lax
from jax.experimental import pallas as pl
from jax.experimental.pallas import tpu as pltpu
from jax.experimental.pallas import tpu_sc as plsc

NUM_VQ = 4
D = 768
L = 16
NC, NS = 2, 16
NW = NC * NS
CHT = 32
CHC = 16


def _sc_kernel(P, V, ids_hbm, mask_hbm, text_hbm, code_hbm, out_hbm,
               ids_v, mask_v, tpos_v, tidx_v, cpos_v, cidx_v,
               tpos_s, tidx_s, cpos_s, cidx_s,
               tbuf, cbuf, sbuf, sem):
    PW = P // NW
    GROUPS = PW // L
    CPL = PW + CHC
    DUMMY = P
    wid = lax.axis_index("s") * NC + lax.axis_index("c")
    base_p = wid * PW
    iota = lax.iota(jnp.int32, L)

    for i in range(NUM_VQ):
        pltpu.sync_copy(ids_hbm.at[pl.ds(i * P + base_p, PW)],
                        ids_v.at[pl.ds(i * PW, PW)])
    pltpu.sync_copy(mask_hbm.at[pl.ds(base_p, PW)], mask_v)

    zeros = jnp.zeros((L,), jnp.int32)
    dummy = jnp.full((L,), DUMMY, jnp.int32)

    def fill_t(i, _):
        tpos_v[pl.ds(i * L, L)] = dummy
        tidx_v[pl.ds(i * L, L)] = zeros
        return 0
    lax.fori_loop(0, (PW + CHT) // L, fill_t, 0)

    def fill_c(i, _):
        cpos_v[pl.ds(i * L, L)] = dummy
        for j in range(NUM_VQ):
            cidx_v[pl.ds(j * CPL + i * L, L)] = zeros
        return 0
    lax.fori_loop(0, CPL // L, fill_c, 0)

    def compact(g, carry):
        nt, nc = carry
        p0 = g * L
        m16 = mask_v[pl.ds(p0, L)]
        tmask = m16 > 0
        cmask = m16 == 0
        gpos = base_p + p0 + iota
        tid16 = ids_v[pl.ds(p0, L)]
        plsc.store_compressed(tidx_v.at[pl.ds(nt, L)], tid16, mask=tmask)
        plsc.store_compressed(tpos_v.at[pl.ds(nt, L)], gpos, mask=tmask)
        plsc.store_compressed(cpos_v.at[pl.ds(nc, L)], gpos, mask=cmask)
        for i in range(NUM_VQ):
            civ = ids_v[pl.ds(i * PW + p0, L)] + i * V
            plsc.store_compressed(cidx_v.at[pl.ds(i * CPL + nc, L)], civ,
                                  mask=cmask)
        tcnt = jnp.sum(m16)
        return nt + tcnt, nc + (L - tcnt)

    nt, nc = lax.fori_loop(0, GROUPS, compact, (jnp.int32(0), jnp.int32(0)))

    def tchunk(k, _):
        off = k * CHT
        for j in range(CHT // L):
            tidx_s[pl.ds(j * L, L)] = tidx_v[pl.ds(off + j * L, L)]
            tpos_s[pl.ds(j * L, L)] = tpos_v[pl.ds(off + j * L, L)]
        pltpu.async_copy(text_hbm.at[tidx_s], tbuf, sem).wait()
        pltpu.async_copy(tbuf, out_hbm.at[tpos_s], sem).wait()
        return 0
    lax.fori_loop(0, (nt + CHT - 1) // CHT, tchunk, 0)

    DL = D // L
    def cchunk(k, _):
        off = k * CHC
        cpos_s[pl.ds(0, L)] = cpos_v[pl.ds(off, L)]
        for i in range(NUM_VQ):
            cidx_s[pl.ds(i * CHC, L)] = cidx_v[pl.ds(i * CPL + off, L)]
        pltpu.async_copy(code_hbm.at[cidx_s], cbuf, sem).wait()

        def sum4(p, _):
            for d in range(DL):
                s = pl.ds(d * L, L)
                sbuf[p, s] = (cbuf[p, s] + cbuf[CHC + p, s]
                              + cbuf[2 * CHC + p, s] + cbuf[3 * CHC + p, s])
            return 0
        lax.fori_loop(0, CHC, sum4, 0)
        pltpu.async_copy(sbuf, out_hbm.at[cpos_s], sem).wait()
        return 0
    lax.fori_loop(0, (nc + CHC - 1) // CHC, cchunk, 0)


def kernel(input_ids, text_mask, emb_text_table, emb_code_tables):
    B, S, _ = input_ids.shape
    P = B * S
    PW = P // NW
    V = emb_code_tables.shape[1]
    ids_t = jnp.transpose(input_ids.reshape(P, NUM_VQ)).reshape(NUM_VQ * P)
    ids_t = ids_t.astype(jnp.int32)
    mask_flat = text_mask.reshape(P).astype(jnp.int32)
    code_flat = emb_code_tables.reshape(NUM_VQ * V, D)

    mesh = plsc.VectorSubcoreMesh(core_axis_name="c", subcore_axis_name="s",
                                  num_cores=NC, num_subcores=NS)
    run = pl.kernel(
        functools.partial(_sc_kernel, P, V),
        out_type=jax.ShapeDtypeStruct((P + 8, D), jnp.float32),
        mesh=mesh,
        compiler_params=pltpu.CompilerParams(needs_layout_passes=False),
        scratch_types=[
            pltpu.VMEM((NUM_VQ * PW,), jnp.int32),
            pltpu.VMEM((PW,), jnp.int32),
            pltpu.VMEM((PW + CHT,), jnp.int32),
            pltpu.VMEM((PW + CHT,), jnp.int32),
            pltpu.VMEM((PW + CHC,), jnp.int32),
            pltpu.VMEM((NUM_VQ * (PW + CHC),), jnp.int32),
            pltpu.VMEM((CHT,), jnp.int32),
            pltpu.VMEM((CHT,), jnp.int32),
            pltpu.VMEM((CHC,), jnp.int32),
            pltpu.VMEM((NUM_VQ * CHC,), jnp.int32),
            pltpu.VMEM((CHT, D), jnp.float32),
            pltpu.VMEM((NUM_VQ * CHC, D), jnp.float32),
            pltpu.VMEM((CHC, D), jnp.float32),
            pltpu.SemaphoreType.DMA,
        ],
    )
    out = run(ids_t, mask_flat, emb_text_table, code_flat)
    return out[:P].reshape(B, S, D)
